# Initial kernel scaffold; baseline (speedup 1.0000x reference)
import jax
import jax.numpy as jnp
from jax.experimental import pallas as pl


def kernel(x, rW1, rb1, rW2, rb2, eW1, eb1, eW2, eb2):
    raise NotImplementedError("write your pallas kernel here")


# speedup vs baseline: 1.0460x; 1.0460x over previous
import functools
import math

import jax
import jax.numpy as jnp
from jax import lax
from jax.experimental import pallas as pl
from jax.experimental.pallas import tpu as pltpu

T, D, E, K, FF = 2048, 768, 64, 2, 1536
RH = 2 * D
C = int(math.ceil(T * K / E * 1.25))
TB = 256


def _gelu(v):
    return 0.5 * v * (1.0 + lax.erf(v * (1.0 / math.sqrt(2.0))))


def _router_body(x_ref, w1_ref, b1_ref, w2_ref, b2_ref,
                 i1_ref, i2_ref, g1_ref, g2_ref):
    xb = x_ref[...]
    rh = _gelu(jnp.dot(xb, w1_ref[...], preferred_element_type=jnp.float32)
               + b1_ref[...])
    logits = (jnp.dot(rh, w2_ref[...], preferred_element_type=jnp.float32)
              + b2_ref[...])
    iota = lax.broadcasted_iota(jnp.int32, (TB, E), 1)
    l1 = jnp.max(logits, axis=-1, keepdims=True)
    a1 = jnp.min(jnp.where(logits == l1, iota, E), axis=-1, keepdims=True)
    masked = jnp.where(iota == a1, -jnp.inf, logits)
    l2 = jnp.max(masked, axis=-1, keepdims=True)
    a2 = jnp.min(jnp.where(masked == l2, iota, E), axis=-1, keepdims=True)
    g1 = 1.0 / (1.0 + jnp.exp(l2 - l1))
    i1_ref[...] = a1[:, 0]
    i2_ref[...] = a2[:, 0]
    g1_ref[...] = g1[:, 0]
    g2_ref[...] = 1.0 - g1[:, 0]


def _router(x, rW1, rb1, rW2, rb2):
    n = T // TB
    out_shapes = (
        jax.ShapeDtypeStruct((T,), jnp.int32),
        jax.ShapeDtypeStruct((T,), jnp.int32),
        jax.ShapeDtypeStruct((T,), jnp.float32),
        jax.ShapeDtypeStruct((T,), jnp.float32),
    )
    return pl.pallas_call(
        _router_body,
        grid=(n,),
        in_specs=[
            pl.BlockSpec((TB, D), lambda i: (i, 0)),
            pl.BlockSpec((D, RH), lambda i: (0, 0)),
            pl.BlockSpec((RH,), lambda i: (0,)),
            pl.BlockSpec((RH, E), lambda i: (0, 0)),
            pl.BlockSpec((E,), lambda i: (0,)),
        ],
        out_specs=(
            pl.BlockSpec((TB,), lambda i: (i,)),
            pl.BlockSpec((TB,), lambda i: (i,)),
            pl.BlockSpec((TB,), lambda i: (i,)),
            pl.BlockSpec((TB,), lambda i: (i,)),
        ),
        out_shape=out_shapes,
    )(x, rW1, rb1, rW2, rb2)


def _ffn_body(xe_ref, w1_ref, b1_ref, w2_ref, b2_ref, gate_ref, ye_ref):
    xb = xe_ref[...]
    h = _gelu(jnp.dot(xb, w1_ref[0], preferred_element_type=jnp.float32)
              + b1_ref[0])
    y = jnp.dot(h, w2_ref[0], preferred_element_type=jnp.float32) + b2_ref[0]
    ye_ref[...] = y * gate_ref[...]


def _ffn(xe, eW1, eb1, eW2, eb2, disp_gate):
    return pl.pallas_call(
        _ffn_body,
        grid=(E,),
        in_specs=[
            pl.BlockSpec((C, D), lambda e: (e, 0)),
            pl.BlockSpec((1, D, FF), lambda e: (e, 0, 0)),
            pl.BlockSpec((1, 1, FF), lambda e: (e, 0, 0)),
            pl.BlockSpec((1, FF, D), lambda e: (e, 0, 0)),
            pl.BlockSpec((1, 1, D), lambda e: (e, 0, 0)),
            pl.BlockSpec((C, 1), lambda e: (e, 0)),
        ],
        out_specs=pl.BlockSpec((C, D), lambda e: (e, 0)),
        out_shape=jax.ShapeDtypeStruct((E * C, D), jnp.float32),
        compiler_params=pltpu.CompilerParams(
            dimension_semantics=("arbitrary",),
        ),
    )(xe, eW1, eb1.reshape(E, 1, FF), eW2, eb2.reshape(E, 1, D),
      disp_gate.reshape(E * C, 1))


def kernel(x, rW1, rb1, rW2, rb2, eW1, eb1, eW2, eb2):
    i1, i2, g1, g2 = _router(x, rW1, rb1, rW2, rb2)
    flat_exp = jnp.stack([i1, i2], axis=1).reshape(-1)
    flat_gate = jnp.stack([g1, g2], axis=1).reshape(-1)
    flat_tok = jnp.repeat(jnp.arange(T, dtype=jnp.int32), K)
    onehot = jax.nn.one_hot(flat_exp, E, dtype=jnp.int32)
    pos = jnp.cumsum(onehot, axis=0) - 1
    pos = jnp.sum(pos * onehot, axis=1)
    keep = pos < C
    slot = jnp.where(keep, flat_exp * C + pos, E * C)
    disp_tok = jnp.zeros((E * C,), dtype=jnp.int32).at[slot].set(
        flat_tok, mode='drop')
    disp_gate = jnp.zeros((E * C,), dtype=x.dtype).at[slot].set(
        flat_gate, mode='drop')
    xe = jnp.take(x, disp_tok, axis=0)
    ye = _ffn(xe, eW1, eb1, eW2, eb2, disp_gate)
    out = jnp.zeros((T, D), dtype=x.dtype).at[disp_tok].add(ye)
    return out


# speedup vs baseline: 1.3673x; 1.3072x over previous
import math

import jax
import jax.numpy as jnp
from jax import lax
from jax.experimental import pallas as pl
from jax.experimental.pallas import tpu as pltpu

T, D, E, K, FF = 2048, 768, 64, 2, 1536
RH = 2 * D
C = int(math.ceil(T * K / E * 1.25))
EC = E * C
TB = 256
SB = 512


def _gelu(v):
    return 0.5 * v * (1.0 + lax.erf(v * (1.0 / math.sqrt(2.0))))


def _router_body(x_ref, w1_ref, b1_ref, w2_ref, b2_ref,
                 i1_ref, i2_ref, g1_ref, g2_ref):
    xb = x_ref[...]
    rh = _gelu(jnp.dot(xb, w1_ref[...], preferred_element_type=jnp.float32)
               + b1_ref[...])
    logits = (jnp.dot(rh, w2_ref[...], preferred_element_type=jnp.float32)
              + b2_ref[...])
    iota = lax.broadcasted_iota(jnp.int32, (TB, E), 1)
    l1 = jnp.max(logits, axis=-1, keepdims=True)
    a1 = jnp.min(jnp.where(logits == l1, iota, E), axis=-1, keepdims=True)
    masked = jnp.where(iota == a1, -jnp.inf, logits)
    l2 = jnp.max(masked, axis=-1, keepdims=True)
    a2 = jnp.min(jnp.where(masked == l2, iota, E), axis=-1, keepdims=True)
    g1 = 1.0 / (1.0 + jnp.exp(l2 - l1))
    i1_ref[...] = a1
    i2_ref[...] = a2
    g1_ref[...] = g1
    g2_ref[...] = 1.0 - g1


def _router(x, rW1, rb1, rW2, rb2):
    n = T // TB
    out_shapes = (
        jax.ShapeDtypeStruct((T, 1), jnp.int32),
        jax.ShapeDtypeStruct((T, 1), jnp.int32),
        jax.ShapeDtypeStruct((T, 1), jnp.float32),
        jax.ShapeDtypeStruct((T, 1), jnp.float32),
    )
    return pl.pallas_call(
        _router_body,
        grid=(n,),
        in_specs=[
            pl.BlockSpec((TB, D), lambda i: (i, 0)),
            pl.BlockSpec((D, RH), lambda i: (0, 0)),
            pl.BlockSpec((RH,), lambda i: (0,)),
            pl.BlockSpec((RH, E), lambda i: (0, 0)),
            pl.BlockSpec((E,), lambda i: (0,)),
        ],
        out_specs=tuple(pl.BlockSpec((TB, 1), lambda i: (i, 0))
                        for _ in range(4)),
        out_shape=out_shapes,
    )(x, rW1, rb1, rW2, rb2)


def _pos_body(i1_ref, i2_ref, s1_ref, s2_ref):
    i1 = i1_ref[...]
    i2 = i2_ref[...]
    iota = lax.broadcasted_iota(jnp.int32, (T, E), 1)
    oh1 = (i1 == iota).astype(jnp.int32)
    oh2 = (i2 == iota).astype(jnp.int32)
    s = oh1 + oh2
    k = 1
    while k < T:
        s = s + jnp.concatenate(
            [jnp.zeros((k, E), jnp.int32), s[:T - k]], axis=0)
        k *= 2
    pos1 = jnp.sum(oh1 * (s - oh1 - oh2), axis=1, keepdims=True)
    pos2 = jnp.sum(oh2 * (s - oh2), axis=1, keepdims=True)
    s1 = jnp.where(pos1 < C, i1 * C + pos1, EC)
    s2 = jnp.where(pos2 < C, i2 * C + pos2, EC)
    s1_ref[...] = s1
    s2_ref[...] = s2


def _positions(i1, i2):
    return pl.pallas_call(
        _pos_body,
        out_shape=(jax.ShapeDtypeStruct((T, 1), jnp.int32),
                   jax.ShapeDtypeStruct((T, 1), jnp.int32)),
    )(i1, i2)


def _disp_body(s1_ref, s2_ref, disp_ref):
    base = pl.program_id(0) * SB
    lane = base + lax.broadcasted_iota(jnp.int32, (1, SB), 1)
    tokf = lax.broadcasted_iota(jnp.int32, (T, 1), 0).astype(jnp.float32)
    m1 = (s1_ref[...] == lane).astype(jnp.float32)
    m2 = (s2_ref[...] == lane).astype(jnp.float32)
    acc = lax.dot_general(tokf, m1, (((0,), (0,)), ((), ())),
                          preferred_element_type=jnp.float32)
    acc += lax.dot_general(tokf, m2, (((0,), (0,)), ((), ())),
                           preferred_element_type=jnp.float32)
    disp_ref[...] = acc


def _disp_table(s1, s2):
    return pl.pallas_call(
        _disp_body,
        grid=(EC // SB,),
        in_specs=[
            pl.BlockSpec((T, 1), lambda i: (0, 0)),
            pl.BlockSpec((T, 1), lambda i: (0, 0)),
        ],
        out_specs=pl.BlockSpec((1, SB), lambda i: (0, i)),
        out_shape=jax.ShapeDtypeStruct((1, EC), jnp.float32),
    )(s1, s2)


def _ffn_body(xe_ref, w1_ref, b1_ref, w2_ref, b2_ref, ye_ref):
    e = pl.program_id(0)

    @pl.when(e < E)
    def _():
        xb = xe_ref[...]
        h = _gelu(jnp.dot(xb, w1_ref[0], preferred_element_type=jnp.float32)
                  + b1_ref[0])
        y = (jnp.dot(h, w2_ref[0], preferred_element_type=jnp.float32)
             + b2_ref[0])
        ye_ref[...] = y

    @pl.when(e >= E)
    def _():
        ye_ref[...] = jnp.zeros((C, D), jnp.float32)


def _ffn(xe, eW1, eb1, eW2, eb2):
    wi = lambda e: (jnp.minimum(e, E - 1), 0, 0)
    return pl.pallas_call(
        _ffn_body,
        grid=(E + 1,),
        in_specs=[
            pl.BlockSpec((C, D), lambda e: (jnp.minimum(e, E - 1), 0)),
            pl.BlockSpec((1, D, FF), wi),
            pl.BlockSpec((1, 1, FF), wi),
            pl.BlockSpec((1, FF, D), wi),
            pl.BlockSpec((1, 1, D), wi),
        ],
        out_specs=pl.BlockSpec((C, D), lambda e: (e, 0)),
        out_shape=jax.ShapeDtypeStruct((EC + C, D), jnp.float32),
        compiler_params=pltpu.CompilerParams(
            dimension_semantics=("arbitrary",),
        ),
    )(xe, eW1, eb1.reshape(E, 1, FF), eW2, eb2.reshape(E, 1, D))


def kernel(x, rW1, rb1, rW2, rb2, eW1, eb1, eW2, eb2):
    i1, i2, g1, g2 = _router(x, rW1, rb1, rW2, rb2)
    s1, s2 = _positions(i1, i2)
    disp_tok = _disp_table(s1, s2).reshape(EC).astype(jnp.int32)
    xe = jnp.take(x, disp_tok, axis=0)
    ye = _ffn(xe, eW1, eb1, eW2, eb2)
    out = (g1 * jnp.take(ye, s1[:, 0], axis=0)
           + g2 * jnp.take(ye, s2[:, 0], axis=0))
    return out
